# Initial kernel scaffold; baseline (speedup 1.0000x reference)
import jax
import jax.numpy as jnp
from jax.experimental import pallas as pl


def kernel(x, edge_index, W_root, W_neigh, b):
    raise NotImplementedError("write your pallas kernel here")


# speedup vs baseline: 5.3894x; 5.3894x over previous
import functools

import jax
import jax.numpy as jnp
from jax import lax
from jax.experimental import pallas as pl
from jax.experimental.pallas import tpu as pltpu
from jax.experimental.pallas import tpu_sc as plsc

N = 10000
E = 320000
D = 128

NC = 2
NS = 16
NW = NC * NS
EDGES_PER_TILE = E // NW
K = 80
CHUNKS = EDGES_PER_TILE // K
ROWS_PER_TILE = 624
TAIL_START = ROWS_PER_TILE * NS
TAIL_ROWS = N - TAIL_START


def _segment_sum_sc(x, src, dst, zeros):
    mesh = plsc.VectorSubcoreMesh(core_axis_name="c", subcore_axis_name="s")

    @functools.partial(
        pl.kernel,
        mesh=mesh,
        out_type=jax.ShapeDtypeStruct((NC, N, D), jnp.float32),
        scratch_types=[
            pltpu.VMEM((K,), jnp.int32),
            pltpu.VMEM((K,), jnp.int32),
            pltpu.VMEM((K, D), jnp.float32),
            pltpu.VMEM_SHARED((N, D), jnp.float32),
            pltpu.SemaphoreType.DMA,
        ],
    )
    def seg(x_hbm, src_hbm, dst_hbm, zeros_hbm, out_hbm,
            src_v, dst_v, rows_v, agg_sh, sem):
        c = lax.axis_index("c")
        s = lax.axis_index("s")
        wid = s * NC + c

        pltpu.sync_copy(zeros_hbm.at[pl.ds(s * ROWS_PER_TILE, ROWS_PER_TILE)],
                        agg_sh.at[pl.ds(s * ROWS_PER_TILE, ROWS_PER_TILE)])

        @pl.when(s == NS - 1)
        def _zero_tail():
            pltpu.sync_copy(zeros_hbm.at[pl.ds(TAIL_START, TAIL_ROWS)],
                            agg_sh.at[pl.ds(TAIL_START, TAIL_ROWS)])

        plsc.subcore_barrier()

        base = wid * EDGES_PER_TILE

        def body(i, carry):
            off = base + i * K
            pltpu.sync_copy(src_hbm.at[pl.ds(off, K)], src_v)
            pltpu.sync_copy(dst_hbm.at[pl.ds(off, K)], dst_v)
            pltpu.async_copy(x_hbm.at[src_v], rows_v, sem).wait()
            pltpu.sync_copy(rows_v, agg_sh.at[dst_v], add=True)
            return carry

        lax.fori_loop(0, CHUNKS, body, 0)
        plsc.subcore_barrier()

        pltpu.sync_copy(agg_sh.at[pl.ds(s * ROWS_PER_TILE, ROWS_PER_TILE)],
                        out_hbm.at[c, pl.ds(s * ROWS_PER_TILE, ROWS_PER_TILE)])

        @pl.when(s == NS - 1)
        def _copy_tail():
            pltpu.sync_copy(agg_sh.at[pl.ds(TAIL_START, TAIL_ROWS)],
                            out_hbm.at[c, pl.ds(TAIL_START, TAIL_ROWS)])

    return seg(x, src, dst, zeros)


BLK = 2000


def _dense_body(x_ref, a0_ref, a1_ref, wr_ref, wn_ref, b_ref, o_ref):
    acc = jnp.dot(x_ref[...], wr_ref[...], preferred_element_type=jnp.float32)
    acc = acc + jnp.dot(a0_ref[...] + a1_ref[...], wn_ref[...],
                        preferred_element_type=jnp.float32)
    o_ref[...] = acc + b_ref[...]


def kernel(x, edge_index, W_root, W_neigh, b):
    src = edge_index[0]
    dst = edge_index[1]
    zeros = jnp.zeros((N, D), jnp.float32)
    parts = _segment_sum_sc(x, src, dst, zeros)
    out = pl.pallas_call(
        _dense_body,
        grid=(N // BLK,),
        in_specs=[
            pl.BlockSpec((BLK, D), lambda i: (i, 0)),
            pl.BlockSpec((BLK, D), lambda i: (i, 0)),
            pl.BlockSpec((BLK, D), lambda i: (i, 0)),
            pl.BlockSpec((D, D), lambda i: (0, 0)),
            pl.BlockSpec((D, D), lambda i: (0, 0)),
            pl.BlockSpec((1, D), lambda i: (0, 0)),
        ],
        out_specs=pl.BlockSpec((BLK, D), lambda i: (i, 0)),
        out_shape=jax.ShapeDtypeStruct((N, D), jnp.float32),
    )(x, parts[0], parts[1], W_root, W_neigh, b.reshape(1, D))
    return out


# speedup vs baseline: 13.5213x; 2.5089x over previous
import functools

import jax
import jax.numpy as jnp
from jax import lax
from jax.experimental import pallas as pl
from jax.experimental.pallas import tpu as pltpu
from jax.experimental.pallas import tpu_sc as plsc

N = 10000
E = 320000
D = 128

NC = 2
NS = 16
NW = NC * NS
EDGES_PER_TILE = E // NW
K = 40
CHUNKS = EDGES_PER_TILE // K
NBUF = 5
OUTER = CHUNKS // NBUF
ROWS_PER_TILE = 624
TAIL_START = ROWS_PER_TILE * NS
TAIL_ROWS = N - TAIL_START


def _build_seg():
    mesh = plsc.VectorSubcoreMesh(core_axis_name="c", subcore_axis_name="s")

    @functools.partial(
        pl.kernel,
        mesh=mesh,
        out_type=jax.ShapeDtypeStruct((NC, N, D), jnp.float32),
        scratch_types=[
            pltpu.VMEM((EDGES_PER_TILE,), jnp.int32),
            pltpu.VMEM((EDGES_PER_TILE,), jnp.int32),
        ] + [pltpu.VMEM((K, D), jnp.float32) for _ in range(NBUF)]
          + [pltpu.SemaphoreType.DMA for _ in range(NBUF)]
          + [pltpu.VMEM_SHARED((N, D), jnp.float32)],
    )
    def seg(x_hbm, src_hbm, dst_hbm, zeros_hbm, out_hbm, *scr):
        src_v, dst_v = scr[0], scr[1]
        rows = scr[2:2 + NBUF]
        sems = scr[2 + NBUF:2 + 2 * NBUF]
        agg_sh = scr[2 + 2 * NBUF]
        c = lax.axis_index("c")
        s = lax.axis_index("s")
        wid = s * NC + c

        pltpu.sync_copy(zeros_hbm.at[pl.ds(s * ROWS_PER_TILE, ROWS_PER_TILE)],
                        agg_sh.at[pl.ds(s * ROWS_PER_TILE, ROWS_PER_TILE)])

        @pl.when(s == NS - 1)
        def _zero_tail():
            pltpu.sync_copy(zeros_hbm.at[pl.ds(TAIL_START, TAIL_ROWS)],
                            agg_sh.at[pl.ds(TAIL_START, TAIL_ROWS)])

        base = wid * EDGES_PER_TILE
        pltpu.sync_copy(src_hbm.at[pl.ds(base, EDGES_PER_TILE)], src_v)
        pltpu.sync_copy(dst_hbm.at[pl.ds(base, EDGES_PER_TILE)], dst_v)
        plsc.subcore_barrier()

        for b in range(NBUF):
            pltpu.async_copy(x_hbm.at[src_v.at[pl.ds(b * K, K)]],
                             rows[b], sems[b])

        def body(j, carry):
            for b in range(NBUF):
                ch = j * NBUF + b
                pltpu.make_async_copy(x_hbm.at[pl.ds(0, K)], rows[b],
                                      sems[b]).wait()
                pltpu.sync_copy(rows[b], agg_sh.at[dst_v.at[pl.ds(ch * K, K)]],
                                add=True)

                @pl.when(j < OUTER - 1)
                def _next():
                    pltpu.async_copy(
                        x_hbm.at[src_v.at[pl.ds((ch + NBUF) * K, K)]],
                        rows[b], sems[b])
            return carry

        lax.fori_loop(0, OUTER, body, 0)
        plsc.subcore_barrier()

        pltpu.sync_copy(agg_sh.at[pl.ds(s * ROWS_PER_TILE, ROWS_PER_TILE)],
                        out_hbm.at[c, pl.ds(s * ROWS_PER_TILE, ROWS_PER_TILE)])

        @pl.when(s == NS - 1)
        def _copy_tail():
            pltpu.sync_copy(agg_sh.at[pl.ds(TAIL_START, TAIL_ROWS)],
                            out_hbm.at[c, pl.ds(TAIL_START, TAIL_ROWS)])

    return seg


_seg_call = _build_seg()

BLK = 2000


def _dense_body(x_ref, a0_ref, a1_ref, wr_ref, wn_ref, b_ref, o_ref):
    acc = jnp.dot(x_ref[...], wr_ref[...], preferred_element_type=jnp.float32)
    acc = acc + jnp.dot(a0_ref[...] + a1_ref[...], wn_ref[...],
                        preferred_element_type=jnp.float32)
    o_ref[...] = acc + b_ref[...]


def kernel(x, edge_index, W_root, W_neigh, b):
    src = edge_index[0]
    dst = edge_index[1]
    zeros = jnp.zeros((N, D), jnp.float32)
    parts = _seg_call(x, src, dst, zeros)
    out = pl.pallas_call(
        _dense_body,
        grid=(N // BLK,),
        in_specs=[
            pl.BlockSpec((BLK, D), lambda i: (i, 0)),
            pl.BlockSpec((BLK, D), lambda i: (i, 0)),
            pl.BlockSpec((BLK, D), lambda i: (i, 0)),
            pl.BlockSpec((D, D), lambda i: (0, 0)),
            pl.BlockSpec((D, D), lambda i: (0, 0)),
            pl.BlockSpec((1, D), lambda i: (0, 0)),
        ],
        out_specs=pl.BlockSpec((BLK, D), lambda i: (i, 0)),
        out_shape=jax.ShapeDtypeStruct((N, D), jnp.float32),
    )(x, parts[0], parts[1], W_root, W_neigh, b.reshape(1, D))
    return out
